# baseline (speedup 1.0000x reference)
import functools

import jax
import jax.numpy as jnp
from jax import lax
from jax.experimental import pallas as pl
from jax.experimental.pallas import tpu as pltpu
from jax.experimental.pallas import tpu_sc as plsc

N = 10000
D = 128
NG = 64
NA = 10
NE = 320000
NTILES = 32
CHUNK = 128
CH = 80
PH_CH = CH // 2
EPAD = NTILES * CH * CHUNK
NPAD = 10112
DUMMY = 10000
RPT = NPAD // 16
BLK = 1000
GRID = N // BLK


def _sc_mesh():
    return plsc.VectorSubcoreMesh(core_axis_name="c", subcore_axis_name="s")


def _sc_hist(dst_r, ones16, zeros16):
    @functools.partial(
        pl.kernel,
        mesh=_sc_mesh(),
        out_type=jax.ShapeDtypeStruct((2, NPAD, 16), jnp.float32),
        scratch_types=[
            pltpu.VMEM((CH, CHUNK), jnp.int32),
            pltpu.VMEM((CHUNK, 16), jnp.float32),
            pltpu.VMEM_SHARED((NPAD, 16), jnp.float32),
        ],
    )
    def k(dst_hbm, ones_hbm, zero_hbm, out_hbm, dst_v, ones_v, deg_sp):
        c = lax.axis_index("c")
        s = lax.axis_index("s")
        wid = s * 2 + c
        r0 = s * RPT
        pltpu.sync_copy(zero_hbm.at[pl.ds(r0, RPT)], deg_sp.at[pl.ds(r0, RPT)])
        pltpu.sync_copy(ones_hbm, ones_v)
        pltpu.sync_copy(dst_hbm.at[wid], dst_v)
        plsc.subcore_barrier()

        def body(j, carry):
            pltpu.sync_copy(ones_v, deg_sp.at[dst_v.at[j]], add=True)
            return carry

        lax.fori_loop(0, CH, body, 0)
        plsc.subcore_barrier()
        pltpu.sync_copy(deg_sp.at[pl.ds(r0, RPT)], out_hbm.at[c, pl.ds(r0, RPT)])

    return k(dst_r, ones16, zeros16)


def _sc_scatter(hs, ei_r, zeros):
    @functools.partial(
        pl.kernel,
        mesh=_sc_mesh(),
        out_type=jax.ShapeDtypeStruct((2, NPAD, D), jnp.float32),
        scratch_types=[
            pltpu.VMEM((PH_CH, 2, CHUNK), jnp.int32),
            pltpu.VMEM((2, CHUNK, D), jnp.float32),
            pltpu.VMEM_SHARED((NPAD, D), jnp.float32),
        ] + [pltpu.SemaphoreType.DMA] * 4,
    )
    def k(hs_hbm, ei_hbm, zero_hbm, out_hbm, idx_v, rows_v, acc_sp, *sems):
        gsems = sems[:2]
        ssems = sems[2:4]
        c = lax.axis_index("c")
        s = lax.axis_index("s")
        wid = s * 2 + c
        r0 = s * RPT
        pltpu.sync_copy(zero_hbm.at[pl.ds(r0, RPT)], acc_sp.at[pl.ds(r0, RPT)])
        plsc.subcore_barrier()

        def _gather(j, b):
            pltpu.async_copy(hs_hbm.at[idx_v.at[j, 0]], rows_v.at[b],
                             gsems[b])

        def _wait_gather(j, b):
            pltpu.make_async_copy(hs_hbm.at[idx_v.at[j, 0]], rows_v.at[b],
                                  gsems[b]).wait()

        def _scatter(j, b):
            pltpu.async_copy(rows_v.at[b], acc_sp.at[idx_v.at[j, 1]],
                             ssems[b], add=True)

        def _wait_scatter(j, b):
            pltpu.make_async_copy(rows_v.at[b], acc_sp.at[idx_v.at[j, 1]],
                                  ssems[b]).wait()

        for ph in range(2):
            pltpu.sync_copy(ei_hbm.at[wid, ph], idx_v)
            _gather(0, 0)

            def body(g, carry):
                for b in range(2):
                    j = g * 2 + b
                    _wait_gather(j, b)
                    _scatter(j, b)
                    if b == 0:
                        @pl.when(g > 0)
                        def _(j=j, b=b):
                            _wait_scatter(j - 1, 1 - b)
                    else:
                        _wait_scatter(j - 1, 1 - b)
                    if b == 1:
                        @pl.when(j + 1 < PH_CH)
                        def _(j=j, b=b):
                            _gather(j + 1, 1 - b)
                    else:
                        _gather(j + 1, 1 - b)
                return carry

            lax.fori_loop(0, PH_CH // 2, body, 0)
            _wait_scatter(PH_CH - 1, 1)
        plsc.subcore_barrier()
        pltpu.sync_copy(acc_sp.at[pl.ds(r0, RPT)], out_hbm.at[c, pl.ds(r0, RPT)])

    return k(hs, ei_r, zeros)


def _dis_from(dacc_ref):
    deg = dacc_ref[0, :, 0:1] + dacc_ref[1, :, 0:1] + 1.0
    return lax.rsqrt(deg)


def _t1_body(x_ref, w_ref, o_ref):
    o_ref[...] = jnp.dot(x_ref[...], w_ref[...], preferred_element_type=jnp.float32)


def _t1b_body(dacc_ref, hm_ref, o_ref):
    o_ref[...] = _dis_from(dacc_ref) * hm_ref[...]


def _t2_body(dacc_ref, acc_ref, hs_ref, w_ref, b_ref, o_ref):
    dis = _dis_from(dacc_ref)
    pre = dis * (acc_ref[0] + acc_ref[1] + hs_ref[...]) + b_ref[...]
    h = jnp.maximum(pre, 0.0)
    o_ref[...] = jnp.dot(dis * h, w_ref[...], preferred_element_type=jnp.float32)


def _t3_body(dacc_ref, acc_ref, hs_ref, batch_ref, b_ref, wh_ref, bh_ref,
             o_ref, sums, counts):
    i = pl.program_id(0)

    @pl.when(i == 0)
    def _():
        sums[...] = jnp.zeros_like(sums)
        counts[...] = jnp.zeros_like(counts)

    dis = _dis_from(dacc_ref)
    h2 = dis * (acc_ref[0] + acc_ref[1] + hs_ref[...]) + b_ref[...]
    bb = batch_ref[0]
    gids = lax.broadcasted_iota(jnp.int32, (NG, BLK), 0)
    p = (bb == gids).astype(jnp.float32)
    sums[...] += jnp.dot(p, h2, preferred_element_type=jnp.float32)
    counts[...] += jnp.sum(p, axis=1, keepdims=True)

    @pl.when(i == pl.num_programs(0) - 1)
    def _():
        pooled = sums[...] / jnp.maximum(counts[...], 1.0)
        o_ref[...] = (jnp.dot(pooled, wh_ref[...],
                              preferred_element_type=jnp.float32) + bh_ref[...])


def _spec_rows(bs):
    return pl.BlockSpec(bs, lambda i: (0, i, 0))


def kernel(x, edge_index, batch, W1, b1, W2, b2, Wh, bh):
    src = edge_index[0].astype(jnp.int32)
    dst = edge_index[1].astype(jnp.int32)
    npad_e = EPAD - NE
    src_r = jnp.concatenate([src, jnp.zeros((npad_e,), jnp.int32)]
                            ).reshape(NTILES, CH, CHUNK)
    dst_r = jnp.concatenate([dst, jnp.full((npad_e,), DUMMY, jnp.int32)]
                            ).reshape(NTILES, CH, CHUNK)
    ei_r = jnp.stack([src_r.reshape(NTILES, 2, PH_CH, CHUNK),
                      dst_r.reshape(NTILES, 2, PH_CH, CHUNK)], axis=3)
    zeros = jnp.zeros((NPAD, D), jnp.float32)
    zeros16 = jnp.zeros((NPAD, 16), jnp.float32)
    ones16 = jnp.ones((CHUNK, 16), jnp.float32)
    batch_r = batch.astype(jnp.int32).reshape(GRID, 1, BLK)
    b1r = b1.reshape(1, D)
    b2r = b2.reshape(1, D)
    whp = jnp.zeros((D, 128), jnp.float32).at[:, :NA].set(Wh)
    bhp = jnp.zeros((1, 128), jnp.float32).at[0, :NA].set(bh)

    degacc = _sc_hist(dst_r, ones16, zeros16)

    hm1 = pl.pallas_call(
        _t1_body,
        grid=(GRID,),
        in_specs=[pl.BlockSpec((BLK, D), lambda i: (i, 0)),
                  pl.BlockSpec((D, D), lambda i: (0, 0))],
        out_specs=pl.BlockSpec((BLK, D), lambda i: (i, 0)),
        out_shape=jax.ShapeDtypeStruct((N, D), jnp.float32),
    )(x, W1)

    hs1 = pl.pallas_call(
        _t1b_body,
        grid=(GRID,),
        in_specs=[_spec_rows((2, BLK, 16)),
                  pl.BlockSpec((BLK, D), lambda i: (i, 0))],
        out_specs=pl.BlockSpec((BLK, D), lambda i: (i, 0)),
        out_shape=jax.ShapeDtypeStruct((N, D), jnp.float32),
    )(degacc, hm1)

    acc1 = _sc_scatter(hs1, ei_r, zeros)

    hs2 = pl.pallas_call(
        _t2_body,
        grid=(GRID,),
        in_specs=[_spec_rows((2, BLK, 16)),
                  _spec_rows((2, BLK, D)),
                  pl.BlockSpec((BLK, D), lambda i: (i, 0)),
                  pl.BlockSpec((D, D), lambda i: (0, 0)),
                  pl.BlockSpec((1, D), lambda i: (0, 0))],
        out_specs=pl.BlockSpec((BLK, D), lambda i: (i, 0)),
        out_shape=jax.ShapeDtypeStruct((N, D), jnp.float32),
    )(degacc, acc1, hs1, W2, b1r)

    acc2 = _sc_scatter(hs2, ei_r, zeros)

    out = pl.pallas_call(
        _t3_body,
        grid=(GRID,),
        in_specs=[_spec_rows((2, BLK, 16)),
                  _spec_rows((2, BLK, D)),
                  pl.BlockSpec((BLK, D), lambda i: (i, 0)),
                  pl.BlockSpec((1, 1, BLK), lambda i: (i, 0, 0)),
                  pl.BlockSpec((1, D), lambda i: (0, 0)),
                  pl.BlockSpec((D, 128), lambda i: (0, 0)),
                  pl.BlockSpec((1, 128), lambda i: (0, 0))],
        out_specs=pl.BlockSpec((NG, 128), lambda i: (0, 0)),
        out_shape=jax.ShapeDtypeStruct((NG, 128), jnp.float32),
        scratch_shapes=[pltpu.VMEM((NG, D), jnp.float32),
                        pltpu.VMEM((NG, 128), jnp.float32)],
    )(degacc, acc2, hs2, batch_r, b2r, whp, bhp)

    return out[:, :NA]


# speedup vs baseline: 2.1636x; 2.1636x over previous
import functools

import jax
import jax.numpy as jnp
from jax import lax
from jax.experimental import pallas as pl
from jax.experimental.pallas import tpu as pltpu
from jax.experimental.pallas import tpu_sc as plsc

N = 10000
D = 128
NG = 64
NA = 10
NE = 320000
NTILES = 32
CHUNK = 128
CH = 80
PH_CH = CH // 2
EPAD = NTILES * CH * CHUNK
NPAD = 10112
DUMMY = 10000
RPT = NPAD // 16
BLK = 1000
GRID = N // BLK


def _sc_mesh():
    return plsc.VectorSubcoreMesh(core_axis_name="c", subcore_axis_name="s")


def _sc_hist(dst_r, ones16, zeros16):
    @functools.partial(
        pl.kernel,
        mesh=_sc_mesh(),
        out_type=jax.ShapeDtypeStruct((2, NPAD, 16), jnp.float32),
        scratch_types=[
            pltpu.VMEM((CH, CHUNK), jnp.int32),
            pltpu.VMEM((CHUNK, 16), jnp.float32),
            pltpu.VMEM_SHARED((NPAD, 16), jnp.float32),
        ],
    )
    def k(dst_hbm, ones_hbm, zero_hbm, out_hbm, dst_v, ones_v, deg_sp):
        c = lax.axis_index("c")
        s = lax.axis_index("s")
        wid = s * 2 + c
        r0 = s * RPT
        pltpu.sync_copy(zero_hbm.at[pl.ds(r0, RPT)], deg_sp.at[pl.ds(r0, RPT)])
        pltpu.sync_copy(ones_hbm, ones_v)
        pltpu.sync_copy(dst_hbm.at[wid], dst_v)
        plsc.subcore_barrier()

        def body(j, carry):
            pltpu.sync_copy(ones_v, deg_sp.at[dst_v.at[j]], add=True)
            return carry

        lax.fori_loop(0, CH, body, 0)
        plsc.subcore_barrier()
        pltpu.sync_copy(deg_sp.at[pl.ds(r0, RPT)], out_hbm.at[c, pl.ds(r0, RPT)])

    return k(dst_r, ones16, zeros16)


def _sc_scatter(hs, src_r, dst_r, zeros):
    @functools.partial(
        pl.kernel,
        mesh=_sc_mesh(),
        out_type=jax.ShapeDtypeStruct((2, NPAD, D), jnp.float32),
        scratch_types=[
            pltpu.VMEM((CH, CHUNK), jnp.int32),
            pltpu.VMEM((CH, CHUNK), jnp.int32),
            pltpu.VMEM((CHUNK, D), jnp.float32),
            pltpu.VMEM_SHARED((NPAD, D), jnp.float32),
        ],
    )
    def k(hs_hbm, src_hbm, dst_hbm, zero_hbm, out_hbm, src_v, dst_v, rows_v, acc_sp):
        c = lax.axis_index("c")
        s = lax.axis_index("s")
        wid = s * 2 + c
        r0 = s * RPT
        pltpu.sync_copy(zero_hbm.at[pl.ds(r0, RPT)], acc_sp.at[pl.ds(r0, RPT)])
        pltpu.sync_copy(src_hbm.at[wid], src_v)
        pltpu.sync_copy(dst_hbm.at[wid], dst_v)
        plsc.subcore_barrier()

        def body(j, carry):
            pltpu.sync_copy(hs_hbm.at[src_v.at[j]], rows_v)
            pltpu.sync_copy(rows_v, acc_sp.at[dst_v.at[j]], add=True)
            return carry

        lax.fori_loop(0, CH, body, 0)
        plsc.subcore_barrier()
        pltpu.sync_copy(acc_sp.at[pl.ds(r0, RPT)], out_hbm.at[c, pl.ds(r0, RPT)])

    return k(hs, src_r, dst_r, zeros)


def _dis_from(dacc_ref):
    deg = dacc_ref[0, :, 0:1] + dacc_ref[1, :, 0:1] + 1.0
    return lax.rsqrt(deg)


def _t1_body(x_ref, w_ref, o_ref):
    o_ref[...] = jnp.dot(x_ref[...], w_ref[...], preferred_element_type=jnp.float32)


def _t1b_body(dacc_ref, hm_ref, o_ref):
    o_ref[...] = _dis_from(dacc_ref) * hm_ref[...]


def _t2_body(dacc_ref, acc_ref, hs_ref, w_ref, b_ref, o_ref):
    dis = _dis_from(dacc_ref)
    pre = dis * (acc_ref[0] + acc_ref[1] + hs_ref[...]) + b_ref[...]
    h = jnp.maximum(pre, 0.0)
    o_ref[...] = jnp.dot(dis * h, w_ref[...], preferred_element_type=jnp.float32)


def _t3_body(dacc_ref, acc_ref, hs_ref, batch_ref, b_ref, wh_ref, bh_ref,
             o_ref, sums, counts):
    i = pl.program_id(0)

    @pl.when(i == 0)
    def _():
        sums[...] = jnp.zeros_like(sums)
        counts[...] = jnp.zeros_like(counts)

    dis = _dis_from(dacc_ref)
    h2 = dis * (acc_ref[0] + acc_ref[1] + hs_ref[...]) + b_ref[...]
    bb = batch_ref[0]
    gids = lax.broadcasted_iota(jnp.int32, (NG, BLK), 0)
    p = (bb == gids).astype(jnp.float32)
    sums[...] += jnp.dot(p, h2, preferred_element_type=jnp.float32)
    counts[...] += jnp.sum(p, axis=1, keepdims=True)

    @pl.when(i == pl.num_programs(0) - 1)
    def _():
        pooled = sums[...] / jnp.maximum(counts[...], 1.0)
        o_ref[...] = (jnp.dot(pooled, wh_ref[...],
                              preferred_element_type=jnp.float32) + bh_ref[...])


def _spec_rows(bs):
    return pl.BlockSpec(bs, lambda i: (0, i, 0))


def kernel(x, edge_index, batch, W1, b1, W2, b2, Wh, bh):
    src = edge_index[0].astype(jnp.int32)
    dst = edge_index[1].astype(jnp.int32)
    npad_e = EPAD - NE
    pad_iota = jnp.arange(npad_e, dtype=jnp.int32)
    src_r = jnp.concatenate([src, pad_iota % N]).reshape(NTILES, CH, CHUNK)
    dst_r = jnp.concatenate([dst, DUMMY + pad_iota % (NPAD - N)]
                            ).reshape(NTILES, CH, CHUNK)
    zeros = jnp.zeros((NPAD, D), jnp.float32)
    zeros16 = jnp.zeros((NPAD, 16), jnp.float32)
    ones16 = jnp.ones((CHUNK, 16), jnp.float32)
    batch_r = batch.astype(jnp.int32).reshape(GRID, 1, BLK)
    b1r = b1.reshape(1, D)
    b2r = b2.reshape(1, D)
    whp = jnp.zeros((D, 128), jnp.float32).at[:, :NA].set(Wh)
    bhp = jnp.zeros((1, 128), jnp.float32).at[0, :NA].set(bh)

    degacc = _sc_hist(dst_r, ones16, zeros16)

    hm1 = pl.pallas_call(
        _t1_body,
        grid=(GRID,),
        in_specs=[pl.BlockSpec((BLK, D), lambda i: (i, 0)),
                  pl.BlockSpec((D, D), lambda i: (0, 0))],
        out_specs=pl.BlockSpec((BLK, D), lambda i: (i, 0)),
        out_shape=jax.ShapeDtypeStruct((N, D), jnp.float32),
    )(x, W1)

    hs1 = pl.pallas_call(
        _t1b_body,
        grid=(GRID,),
        in_specs=[_spec_rows((2, BLK, 16)),
                  pl.BlockSpec((BLK, D), lambda i: (i, 0))],
        out_specs=pl.BlockSpec((BLK, D), lambda i: (i, 0)),
        out_shape=jax.ShapeDtypeStruct((N, D), jnp.float32),
    )(degacc, hm1)

    acc1 = _sc_scatter(hs1, src_r, dst_r, zeros)

    hs2 = pl.pallas_call(
        _t2_body,
        grid=(GRID,),
        in_specs=[_spec_rows((2, BLK, 16)),
                  _spec_rows((2, BLK, D)),
                  pl.BlockSpec((BLK, D), lambda i: (i, 0)),
                  pl.BlockSpec((D, D), lambda i: (0, 0)),
                  pl.BlockSpec((1, D), lambda i: (0, 0))],
        out_specs=pl.BlockSpec((BLK, D), lambda i: (i, 0)),
        out_shape=jax.ShapeDtypeStruct((N, D), jnp.float32),
    )(degacc, acc1, hs1, W2, b1r)

    acc2 = _sc_scatter(hs2, src_r, dst_r, zeros)

    out = pl.pallas_call(
        _t3_body,
        grid=(GRID,),
        in_specs=[_spec_rows((2, BLK, 16)),
                  _spec_rows((2, BLK, D)),
                  pl.BlockSpec((BLK, D), lambda i: (i, 0)),
                  pl.BlockSpec((1, 1, BLK), lambda i: (i, 0, 0)),
                  pl.BlockSpec((1, D), lambda i: (0, 0)),
                  pl.BlockSpec((D, 128), lambda i: (0, 0)),
                  pl.BlockSpec((1, 128), lambda i: (0, 0))],
        out_specs=pl.BlockSpec((NG, 128), lambda i: (0, 0)),
        out_shape=jax.ShapeDtypeStruct((NG, 128), jnp.float32),
        scratch_shapes=[pltpu.VMEM((NG, D), jnp.float32),
                        pltpu.VMEM((NG, 128), jnp.float32)],
    )(degacc, acc2, hs2, batch_r, b2r, whp, bhp)

    return out[:, :NA]


# speedup vs baseline: 2.5379x; 1.1730x over previous
import functools

import jax
import jax.numpy as jnp
from jax import lax
from jax.experimental import pallas as pl
from jax.experimental.pallas import tpu as pltpu
from jax.experimental.pallas import tpu_sc as plsc

N = 10000
D = 128
NG = 64
NA = 10
NE = 320000
NTILES = 32
CHUNK = 128
CH = 80
PH_CH = CH // 2
EPAD = NTILES * CH * CHUNK
NPAD = 10112
DUMMY = 10000
RPT = NPAD // 16
BLK = 1000
GRID = N // BLK


def _sc_mesh():
    return plsc.VectorSubcoreMesh(core_axis_name="c", subcore_axis_name="s")


def _sc_hist(dst_r, ones16, zeros16):
    @functools.partial(
        pl.kernel,
        mesh=_sc_mesh(),
        out_type=jax.ShapeDtypeStruct((2, NPAD, 16), jnp.float32),
        scratch_types=[
            pltpu.VMEM((CH, CHUNK), jnp.int32),
            pltpu.VMEM((CHUNK, 16), jnp.float32),
            pltpu.VMEM_SHARED((NPAD, 16), jnp.float32),
        ],
    )
    def k(dst_hbm, ones_hbm, zero_hbm, out_hbm, dst_v, ones_v, deg_sp):
        c = lax.axis_index("c")
        s = lax.axis_index("s")
        wid = s * 2 + c
        r0 = s * RPT
        pltpu.sync_copy(zero_hbm.at[pl.ds(r0, RPT)], deg_sp.at[pl.ds(r0, RPT)])
        pltpu.sync_copy(ones_hbm, ones_v)
        pltpu.sync_copy(dst_hbm.at[wid], dst_v)
        plsc.subcore_barrier()

        def body(j, carry):
            pltpu.sync_copy(ones_v, deg_sp.at[dst_v.at[j]], add=True)
            return carry

        lax.fori_loop(0, CH, body, 0)
        plsc.subcore_barrier()
        pltpu.sync_copy(deg_sp.at[pl.ds(r0, RPT)], out_hbm.at[c, pl.ds(r0, RPT)])

    return k(dst_r, ones16, zeros16)


def _sc_scatter(hs, src_r, dst_r, zeros):
    @functools.partial(
        pl.kernel,
        mesh=_sc_mesh(),
        out_type=jax.ShapeDtypeStruct((2, NPAD, D), jnp.float32),
        scratch_types=[
            pltpu.VMEM((PH_CH, CHUNK), jnp.int32),
            pltpu.VMEM((PH_CH, CHUNK), jnp.int32),
            pltpu.VMEM((2, CHUNK, D), jnp.float32),
            pltpu.VMEM_SHARED((NPAD, D), jnp.float32),
        ] + [pltpu.SemaphoreType.DMA] * 2,
    )
    def k(hs_hbm, src_hbm, dst_hbm, zero_hbm, out_hbm,
          src_v, dst_v, rows_v, acc_sp, *ssems):
        c = lax.axis_index("c")
        s = lax.axis_index("s")
        wid = s * 2 + c
        r0 = s * RPT
        pltpu.sync_copy(zero_hbm.at[pl.ds(r0, RPT)], acc_sp.at[pl.ds(r0, RPT)])
        plsc.subcore_barrier()

        def _chunk(j, b):
            pltpu.sync_copy(hs_hbm.at[src_v.at[j]], rows_v.at[b])
            return pltpu.async_copy(rows_v.at[b], acc_sp.at[dst_v.at[j]],
                                    ssems[b], add=True)

        for ph in range(2):
            pltpu.sync_copy(src_hbm.at[wid, pl.ds(ph * PH_CH, PH_CH)], src_v)
            pltpu.sync_copy(dst_hbm.at[wid, pl.ds(ph * PH_CH, PH_CH)], dst_v)

            def body(g, carry):
                base = g * 4
                s0 = _chunk(base + 0, 0)
                s1 = _chunk(base + 1, 1)
                s0.wait()
                s0 = _chunk(base + 2, 0)
                s1.wait()
                s1 = _chunk(base + 3, 1)
                s0.wait()
                s1.wait()
                return carry

            lax.fori_loop(0, PH_CH // 4, body, 0)
        plsc.subcore_barrier()
        pltpu.sync_copy(acc_sp.at[pl.ds(r0, RPT)], out_hbm.at[c, pl.ds(r0, RPT)])

    return k(hs, src_r, dst_r, zeros)


def _dis_from(dacc_ref):
    deg = dacc_ref[0, :, 0:1] + dacc_ref[1, :, 0:1] + 1.0
    return lax.rsqrt(deg)


def _t1_body(x_ref, w_ref, o_ref):
    o_ref[...] = jnp.dot(x_ref[...], w_ref[...], preferred_element_type=jnp.float32)


def _t1b_body(dacc_ref, hm_ref, o_ref):
    o_ref[...] = _dis_from(dacc_ref) * hm_ref[...]


def _t2_body(dacc_ref, acc_ref, hs_ref, w_ref, b_ref, o_ref):
    dis = _dis_from(dacc_ref)
    pre = dis * (acc_ref[0] + acc_ref[1] + hs_ref[...]) + b_ref[...]
    h = jnp.maximum(pre, 0.0)
    o_ref[...] = jnp.dot(dis * h, w_ref[...], preferred_element_type=jnp.float32)


def _t3_body(dacc_ref, acc_ref, hs_ref, batch_ref, b_ref, wh_ref, bh_ref,
             o_ref, sums, counts):
    i = pl.program_id(0)

    @pl.when(i == 0)
    def _():
        sums[...] = jnp.zeros_like(sums)
        counts[...] = jnp.zeros_like(counts)

    dis = _dis_from(dacc_ref)
    h2 = dis * (acc_ref[0] + acc_ref[1] + hs_ref[...]) + b_ref[...]
    bb = batch_ref[0]
    gids = lax.broadcasted_iota(jnp.int32, (NG, BLK), 0)
    p = (bb == gids).astype(jnp.float32)
    sums[...] += jnp.dot(p, h2, preferred_element_type=jnp.float32)
    counts[...] += jnp.sum(p, axis=1, keepdims=True)

    @pl.when(i == pl.num_programs(0) - 1)
    def _():
        pooled = sums[...] / jnp.maximum(counts[...], 1.0)
        o_ref[...] = (jnp.dot(pooled, wh_ref[...],
                              preferred_element_type=jnp.float32) + bh_ref[...])


def _spec_rows(bs):
    return pl.BlockSpec(bs, lambda i: (0, i, 0))


def kernel(x, edge_index, batch, W1, b1, W2, b2, Wh, bh):
    src = edge_index[0].astype(jnp.int32)
    dst = edge_index[1].astype(jnp.int32)
    npad_e = EPAD - NE
    pad_iota = jnp.arange(npad_e, dtype=jnp.int32)
    src_r = jnp.concatenate([src, pad_iota % N]).reshape(NTILES, CH, CHUNK)
    dst_r = jnp.concatenate([dst, DUMMY + pad_iota % (NPAD - N)]
                            ).reshape(NTILES, CH, CHUNK)
    zeros = jnp.zeros((NPAD, D), jnp.float32)
    zeros16 = jnp.zeros((NPAD, 16), jnp.float32)
    ones16 = jnp.ones((CHUNK, 16), jnp.float32)
    batch_r = batch.astype(jnp.int32).reshape(GRID, 1, BLK)
    b1r = b1.reshape(1, D)
    b2r = b2.reshape(1, D)
    whp = jnp.zeros((D, 128), jnp.float32).at[:, :NA].set(Wh)
    bhp = jnp.zeros((1, 128), jnp.float32).at[0, :NA].set(bh)

    degacc = _sc_hist(dst_r, ones16, zeros16)

    hm1 = pl.pallas_call(
        _t1_body,
        grid=(GRID,),
        in_specs=[pl.BlockSpec((BLK, D), lambda i: (i, 0)),
                  pl.BlockSpec((D, D), lambda i: (0, 0))],
        out_specs=pl.BlockSpec((BLK, D), lambda i: (i, 0)),
        out_shape=jax.ShapeDtypeStruct((N, D), jnp.float32),
    )(x, W1)

    hs1 = pl.pallas_call(
        _t1b_body,
        grid=(GRID,),
        in_specs=[_spec_rows((2, BLK, 16)),
                  pl.BlockSpec((BLK, D), lambda i: (i, 0))],
        out_specs=pl.BlockSpec((BLK, D), lambda i: (i, 0)),
        out_shape=jax.ShapeDtypeStruct((N, D), jnp.float32),
    )(degacc, hm1)

    acc1 = _sc_scatter(hs1, src_r, dst_r, zeros)

    hs2 = pl.pallas_call(
        _t2_body,
        grid=(GRID,),
        in_specs=[_spec_rows((2, BLK, 16)),
                  _spec_rows((2, BLK, D)),
                  pl.BlockSpec((BLK, D), lambda i: (i, 0)),
                  pl.BlockSpec((D, D), lambda i: (0, 0)),
                  pl.BlockSpec((1, D), lambda i: (0, 0))],
        out_specs=pl.BlockSpec((BLK, D), lambda i: (i, 0)),
        out_shape=jax.ShapeDtypeStruct((N, D), jnp.float32),
    )(degacc, acc1, hs1, W2, b1r)

    acc2 = _sc_scatter(hs2, src_r, dst_r, zeros)

    out = pl.pallas_call(
        _t3_body,
        grid=(GRID,),
        in_specs=[_spec_rows((2, BLK, 16)),
                  _spec_rows((2, BLK, D)),
                  pl.BlockSpec((BLK, D), lambda i: (i, 0)),
                  pl.BlockSpec((1, 1, BLK), lambda i: (i, 0, 0)),
                  pl.BlockSpec((1, D), lambda i: (0, 0)),
                  pl.BlockSpec((D, 128), lambda i: (0, 0)),
                  pl.BlockSpec((1, 128), lambda i: (0, 0))],
        out_specs=pl.BlockSpec((NG, 128), lambda i: (0, 0)),
        out_shape=jax.ShapeDtypeStruct((NG, 128), jnp.float32),
        scratch_shapes=[pltpu.VMEM((NG, D), jnp.float32),
                        pltpu.VMEM((NG, 128), jnp.float32)],
    )(degacc, acc2, hs2, batch_r, b2r, whp, bhp)

    return out[:, :NA]
